# baseline (speedup 1.0000x reference)
import functools

import jax
import jax.numpy as jnp
from jax import lax
from jax.experimental import pallas as pl
from jax.experimental.pallas import tpu as pltpu
from jax.experimental.pallas import tpu_sc as plsc

_N, _K, _D = 10000, 32, 128
_ALPHA = 0.1
_LANES = 16
_CREG = _D // _LANES
_W = _D // 2 // _LANES
_B = 8
_RPB = _B * _K
_IDX_CHUNK = 128
_NCHUNK = _RPB // _IDX_CHUNK
_NBLK = _N // _B
_NW = 32
_BASE = _NBLK // _NW
_EXTRA = _NBLK % _NW
_NBUF = 3


def _sc_body(x_hbm, idx_hbm, h_hbm, out_hbm,
             idx_v, rows_v, h_v, out_v,
             sem_g0, sem_g1, sem_g2, sem_h0, sem_h1, sem_h2):
    wid = lax.axis_index("s") * 2 + lax.axis_index("c")
    nb = _BASE + jnp.where(wid < _EXTRA, 1, 0)
    sb = wid * _BASE + jnp.minimum(wid, _EXTRA)
    end = sb + nb

    sem_g = (sem_g0, sem_g1, sem_g2)
    sem_h = (sem_h0, sem_h1, sem_h2)

    def issue(g, slot):
        pltpu.sync_copy(idx_hbm.at[pl.ds(g * _NCHUNK, _NCHUNK)], idx_v.at[slot])
        pltpu.async_copy(h_hbm.at[pl.ds(g * _B, _B)], h_v.at[slot], sem_h[slot])
        for j in range(_NCHUNK):
            pltpu.async_copy(
                x_hbm.at[idx_v.at[slot, j]],
                rows_v.at[slot, pl.ds(j * _IDX_CHUNK, _IDX_CHUNK)],
                sem_g[slot])

    def drain(g, slot):
        for j in range(_NCHUNK):
            pltpu.make_async_copy(
                x_hbm.at[idx_v.at[slot, j]],
                rows_v.at[slot, pl.ds(j * _IDX_CHUNK, _IDX_CHUNK)],
                sem_g[slot]).wait()
        pltpu.make_async_copy(
            h_hbm.at[pl.ds(g * _B, _B)], h_v.at[slot], sem_h[slot]).wait()

    def unpack2(word):
        lo = plsc.bitcast(lax.shift_left(word, 16), jnp.float32)
        hi = plsc.bitcast(word, jnp.float32)
        return lo, hi

    def compute(g, slot):
        for b in range(_B):
            base_r = b * _K
            acc0 = []
            for w in range(_W):
                lo, hi = unpack2(rows_v[slot, base_r, pl.ds(w * _LANES, _LANES)])
                acc0 += [lo, hi]

            def body(k2, acc, base_r=base_r, slot=slot):
                new = []
                for w in range(_W):
                    lo0, hi0 = unpack2(
                        rows_v[slot, base_r + 2 * k2 - 1,
                               pl.ds(w * _LANES, _LANES)])
                    lo1, hi1 = unpack2(
                        rows_v[slot, base_r + 2 * k2,
                               pl.ds(w * _LANES, _LANES)])
                    new += [acc[2 * w] + (lo0 + lo1),
                            acc[2 * w + 1] + (hi0 + hi1)]
                return tuple(new)

            acc = lax.fori_loop(1, _K // 2, body, tuple(acc0))
            fin = []
            for w in range(_W):
                lo, hi = unpack2(
                    rows_v[slot, base_r + _K - 1, pl.ds(w * _LANES, _LANES)])
                fin += [lo, hi]
            for c in range(2 * _W):
                out_v[b, pl.ds(c * _LANES, _LANES)] = (
                    (1.0 - _ALPHA) * (acc[c] + fin[c])
                    + _ALPHA * h_v[slot, b, pl.ds(c * _LANES, _LANES)])
        pltpu.sync_copy(out_v, out_hbm.at[pl.ds(g * _B, _B)])

    for slot in range(_NBUF):
        @pl.when(slot < nb)
        def _(slot=slot):
            issue(sb + slot, slot)

    def outer(i, carry):
        for slot in range(_NBUF):
            g = sb + _NBUF * i + slot

            @pl.when(g < end)
            def _(g=g, slot=slot):
                drain(g, slot)
                compute(g, slot)

                @pl.when(g + _NBUF < end)
                def _(g=g, slot=slot):
                    issue(g + _NBUF, slot)

        return carry

    lax.fori_loop(0, (nb + _NBUF - 1) // _NBUF, outer, 0)


_sc_call = functools.partial(
    pl.kernel,
    out_type=jax.ShapeDtypeStruct((_N, _D), jnp.float32),
    mesh=plsc.VectorSubcoreMesh(core_axis_name="c", subcore_axis_name="s"),
    compiler_params=pltpu.CompilerParams(needs_layout_passes=False,
                                         use_tc_tiling_on_sc=False),
    scratch_types=[
        pltpu.VMEM((_NBUF, _NCHUNK, _IDX_CHUNK), jnp.int32),
        pltpu.VMEM((_NBUF, _RPB, _D // 2), jnp.int32),
        pltpu.VMEM((_NBUF, _B, _D), jnp.float32),
        pltpu.VMEM((_B, _D), jnp.float32),
        pltpu.SemaphoreType.DMA,
        pltpu.SemaphoreType.DMA,
        pltpu.SemaphoreType.DMA,
        pltpu.SemaphoreType.DMA,
        pltpu.SemaphoreType.DMA,
        pltpu.SemaphoreType.DMA,
    ],
)(_sc_body)


def kernel(x, neighbor_agg, h, neighbor):
    xp = lax.bitcast_convert_type(
        x.astype(jnp.bfloat16).reshape(_N, _D // 2, 2), jnp.int32)
    idx2d = neighbor_agg.astype(jnp.int32).reshape(_N * _K // _IDX_CHUNK,
                                                   _IDX_CHUNK)
    h_p = h.reshape(_N, _W, _LANES, 2).transpose(0, 1, 3, 2).reshape(_N, _D)
    out_p = _sc_call(xp, idx2d, h_p)
    out = out_p.reshape(_N, _W, 2, _LANES).transpose(0, 1, 3, 2).reshape(_N, _D)
    return (out, neighbor)


# speedup vs baseline: 1.0370x; 1.0370x over previous
import functools

import jax
import jax.numpy as jnp
from jax import lax
from jax.experimental import pallas as pl
from jax.experimental.pallas import tpu as pltpu
from jax.experimental.pallas import tpu_sc as plsc

_N, _K, _D = 10000, 32, 128
_ALPHA = 0.1
_LANES = 16
_CREG = _D // _LANES
_W = _D // 2 // _LANES
_B = 8
_RPB = _B * _K
_IDX_CHUNK = 128
_NCHUNK = _RPB // _IDX_CHUNK
_NBLK = _N // _B
_NW = 32
_BASE = _NBLK // _NW
_EXTRA = _NBLK % _NW
_NBUF = 3


def _sc_body(x_hbm, idx_hbm, h_hbm, out_hbm,
             idx_v, rows_v, h_v, out_v,
             sem_g0, sem_g1, sem_g2, sem_h0, sem_h1, sem_h2):
    wid = lax.axis_index("s") * 2 + lax.axis_index("c")
    nb = _BASE + jnp.where(wid < _EXTRA, 1, 0)
    sb = wid * _BASE + jnp.minimum(wid, _EXTRA)
    end = sb + nb

    sem_g = (sem_g0, sem_g1, sem_g2)
    sem_h = (sem_h0, sem_h1, sem_h2)

    def issue(g, slot):
        pltpu.sync_copy(idx_hbm.at[pl.ds(g * _NCHUNK, _NCHUNK)], idx_v.at[slot])
        pltpu.async_copy(h_hbm.at[pl.ds(g * _B, _B)], h_v.at[slot], sem_h[slot])
        for j in range(_NCHUNK):
            pltpu.async_copy(
                x_hbm.at[idx_v.at[slot, j]],
                rows_v.at[slot, pl.ds(j * _IDX_CHUNK, _IDX_CHUNK)],
                sem_g[slot])

    def drain(g, slot):
        for j in range(_NCHUNK):
            pltpu.make_async_copy(
                x_hbm.at[idx_v.at[slot, j]],
                rows_v.at[slot, pl.ds(j * _IDX_CHUNK, _IDX_CHUNK)],
                sem_g[slot]).wait()
        pltpu.make_async_copy(
            h_hbm.at[pl.ds(g * _B, _B)], h_v.at[slot], sem_h[slot]).wait()

    def unpack2(word):
        lo = plsc.bitcast(lax.shift_left(word, 16), jnp.float32)
        hi = plsc.bitcast(word, jnp.float32)
        return lo, hi

    def compute(g, slot):
        for b in range(_B):
            base_r = b * _K
            acc0 = []
            for w in range(_W):
                lo, hi = unpack2(rows_v[slot, base_r, pl.ds(w * _LANES, _LANES)])
                acc0 += [lo, hi]

            def body(k2, acc, base_r=base_r, slot=slot):
                new = []
                for w in range(_W):
                    lo0, hi0 = unpack2(
                        rows_v[slot, base_r + 2 * k2 - 1,
                               pl.ds(w * _LANES, _LANES)])
                    lo1, hi1 = unpack2(
                        rows_v[slot, base_r + 2 * k2,
                               pl.ds(w * _LANES, _LANES)])
                    new += [acc[2 * w] + (lo0 + lo1),
                            acc[2 * w + 1] + (hi0 + hi1)]
                return tuple(new)

            acc = lax.fori_loop(1, _K // 2, body, tuple(acc0))
            fin = []
            for w in range(_W):
                lo, hi = unpack2(
                    rows_v[slot, base_r + _K - 1, pl.ds(w * _LANES, _LANES)])
                fin += [lo, hi]
            for w in range(_W):
                lo_h, hi_h = unpack2(h_v[slot, b, pl.ds(w * _LANES, _LANES)])
                lo_o = (1.0 - _ALPHA) * (acc[2 * w] + fin[2 * w]) + _ALPHA * lo_h
                hi_o = ((1.0 - _ALPHA) * (acc[2 * w + 1] + fin[2 * w + 1])
                        + _ALPHA * hi_h)
                out_v[b, pl.ds(w * 2 * _LANES, 2 * _LANES)] = plsc.pack(
                    lo_o, hi_o, format=plsc.PackFormat.INTERLEAVED)
        pltpu.sync_copy(out_v, out_hbm.at[pl.ds(g * _B, _B)])

    for slot in range(_NBUF):
        @pl.when(slot < nb)
        def _(slot=slot):
            issue(sb + slot, slot)

    def outer(i, carry):
        for slot in range(_NBUF):
            g = sb + _NBUF * i + slot

            @pl.when(g < end)
            def _(g=g, slot=slot):
                drain(g, slot)
                compute(g, slot)

                @pl.when(g + _NBUF < end)
                def _(g=g, slot=slot):
                    issue(g + _NBUF, slot)

        return carry

    lax.fori_loop(0, (nb + _NBUF - 1) // _NBUF, outer, 0)


_sc_call = functools.partial(
    pl.kernel,
    out_type=jax.ShapeDtypeStruct((_N, _D), jnp.bfloat16),
    mesh=plsc.VectorSubcoreMesh(core_axis_name="c", subcore_axis_name="s"),
    compiler_params=pltpu.CompilerParams(needs_layout_passes=False,
                                         use_tc_tiling_on_sc=False),
    scratch_types=[
        pltpu.VMEM((_NBUF, _NCHUNK, _IDX_CHUNK), jnp.int32),
        pltpu.VMEM((_NBUF, _RPB, _D // 2), jnp.int32),
        pltpu.VMEM((_NBUF, _B, _D // 2), jnp.int32),
        pltpu.VMEM((_B, _D), jnp.bfloat16),
        pltpu.SemaphoreType.DMA,
        pltpu.SemaphoreType.DMA,
        pltpu.SemaphoreType.DMA,
        pltpu.SemaphoreType.DMA,
        pltpu.SemaphoreType.DMA,
        pltpu.SemaphoreType.DMA,
    ],
)(_sc_body)


def kernel(x, neighbor_agg, h, neighbor):
    xp = lax.bitcast_convert_type(
        x.astype(jnp.bfloat16).reshape(_N, _D // 2, 2), jnp.int32)
    idx2d = neighbor_agg.astype(jnp.int32).reshape(_N * _K // _IDX_CHUNK,
                                                   _IDX_CHUNK)
    hp = lax.bitcast_convert_type(
        h.astype(jnp.bfloat16).reshape(_N, _D // 2, 2), jnp.int32)
    out = _sc_call(xp, idx2d, hp).astype(jnp.float32)
    return (out, neighbor)


# speedup vs baseline: 1.5414x; 1.4864x over previous
import functools

import jax
import jax.numpy as jnp
from jax import lax
from jax.experimental import pallas as pl
from jax.experimental.pallas import tpu as pltpu
from jax.experimental.pallas import tpu_sc as plsc

_N, _K, _D = 10000, 32, 128
_ALPHA = 0.1
_LANES = 16
_CREG = _D // _LANES
_B = 8
_RPB = _B * _K
_IDX_CHUNK = 128
_NCHUNK = _RPB // _IDX_CHUNK
_NBLK = _N // _B
_NW = 32
_BASE = _NBLK // _NW
_EXTRA = _NBLK % _NW
_NBUF = 3


def _sc_body(x_hbm, idx_hbm, h_hbm, out_hbm,
             idx_v, rows_v, h_v, out_v,
             sem_g0, sem_g1, sem_g2, sem_h0, sem_h1, sem_h2):
    wid = lax.axis_index("s") * 2 + lax.axis_index("c")
    nb = _BASE + jnp.where(wid < _EXTRA, 1, 0)
    sb = wid * _BASE + jnp.minimum(wid, _EXTRA)
    end = sb + nb

    sem_g = (sem_g0, sem_g1, sem_g2)
    sem_h = (sem_h0, sem_h1, sem_h2)

    def issue(g, slot):
        pltpu.sync_copy(idx_hbm.at[pl.ds(g * _NCHUNK, _NCHUNK)], idx_v.at[slot])
        pltpu.async_copy(h_hbm.at[pl.ds(g * _B, _B)], h_v.at[slot], sem_h[slot])
        for j in range(_NCHUNK):
            pltpu.async_copy(
                x_hbm.at[idx_v.at[slot, j]],
                rows_v.at[slot, pl.ds(j * _IDX_CHUNK, _IDX_CHUNK)],
                sem_g[slot])

    def drain(g, slot):
        for j in range(_NCHUNK):
            pltpu.make_async_copy(
                x_hbm.at[idx_v.at[slot, j]],
                rows_v.at[slot, pl.ds(j * _IDX_CHUNK, _IDX_CHUNK)],
                sem_g[slot]).wait()
        pltpu.make_async_copy(
            h_hbm.at[pl.ds(g * _B, _B)], h_v.at[slot], sem_h[slot]).wait()

    def compute(g, slot):
        for b in range(_B):
            base_r = b * _K
            acc0 = tuple(rows_v[slot, base_r, pl.ds(c * _LANES, _LANES)]
                         for c in range(_CREG))

            def body(k, acc, base_r=base_r, slot=slot):
                return tuple(
                    acc[c] + rows_v[slot, base_r + k, pl.ds(c * _LANES, _LANES)]
                    for c in range(_CREG))

            acc = lax.fori_loop(1, _K, body, acc0)
            for c in range(_CREG):
                out_v[b, pl.ds(c * _LANES, _LANES)] = (
                    (1.0 - _ALPHA) * acc[c]
                    + _ALPHA * h_v[slot, b, pl.ds(c * _LANES, _LANES)])
        pltpu.sync_copy(out_v, out_hbm.at[pl.ds(g * _B, _B)])

    for slot in range(_NBUF):
        @pl.when(slot < nb)
        def _(slot=slot):
            issue(sb + slot, slot)

    def outer(i, carry):
        for slot in range(_NBUF):
            g = sb + _NBUF * i + slot

            @pl.when(g < end)
            def _(g=g, slot=slot):
                drain(g, slot)
                compute(g, slot)

                @pl.when(g + _NBUF < end)
                def _(g=g, slot=slot):
                    issue(g + _NBUF, slot)

        return carry

    lax.fori_loop(0, (nb + _NBUF - 1) // _NBUF, outer, 0)


_sc_call = functools.partial(
    pl.kernel,
    out_type=jax.ShapeDtypeStruct((_N, _D), jnp.float32),
    mesh=plsc.VectorSubcoreMesh(core_axis_name="c", subcore_axis_name="s"),
    scratch_types=[
        pltpu.VMEM((_NBUF, _NCHUNK, _IDX_CHUNK), jnp.int32),
        pltpu.VMEM((_NBUF, _RPB, _D), jnp.float32),
        pltpu.VMEM((_NBUF, _B, _D), jnp.float32),
        pltpu.VMEM((_B, _D), jnp.float32),
        pltpu.SemaphoreType.DMA,
        pltpu.SemaphoreType.DMA,
        pltpu.SemaphoreType.DMA,
        pltpu.SemaphoreType.DMA,
        pltpu.SemaphoreType.DMA,
        pltpu.SemaphoreType.DMA,
    ],
)(_sc_body)


def kernel(x, neighbor_agg, h, neighbor):
    idx2d = neighbor_agg.astype(jnp.int32).reshape(_N * _K // _IDX_CHUNK,
                                                   _IDX_CHUNK)
    out = _sc_call(x, idx2d, h)
    return (out, neighbor)
